# Initial kernel scaffold; baseline (speedup 1.0000x reference)
import functools

import jax
import jax.numpy as jnp
from jax import lax
from jax.experimental import pallas as pl
from jax.experimental.pallas import tpu as pltpu
from jax.experimental.pallas import tpu_sc as plsc

EPS = 1e-5
NCORES = 2
NTILES = 16
KB = 80


def _mesh():
    return plsc.VectorSubcoreMesh(
        core_axis_name="c", subcore_axis_name="s",
        num_cores=NCORES, num_subcores=NTILES)


def _make_deg_kernel(n, e):
    epc = e // NCORES
    ept = epc // NTILES
    nb = ept // KB
    assert nb * KB == ept
    rows = n // NTILES
    assert rows * NTILES == n

    def body(dst_hbm, zero_hbm, out_hbm, didx, ones_v, obuf, acc, sem):
        cid = lax.axis_index("c")
        sid = lax.axis_index("s")
        for i in range(KB):
            ones_v[i, :] = jnp.ones((16,), jnp.float32)

        @pl.when(sid == 0)
        def _():
            pltpu.sync_copy(zero_hbm, acc)
        plsc.subcore_barrier()

        base0 = cid * epc + sid * ept

        def step(b, carry):
            base = pl.multiple_of(base0 + b * KB, 8)
            pltpu.sync_copy(dst_hbm.at[pl.ds(base, KB)], didx)
            pltpu.sync_copy(ones_v, acc.at[didx], add=True)
            return carry

        lax.fori_loop(0, nb, step, 0)
        plsc.subcore_barrier()

        r0 = sid * rows
        pltpu.sync_copy(acc.at[pl.ds(r0, rows)], obuf)
        pltpu.sync_copy(obuf, out_hbm.at[cid, pl.ds(r0, rows)])

    return pl.kernel(
        body,
        out_type=jax.ShapeDtypeStruct((NCORES, n, 16), jnp.float32),
        mesh=_mesh(),
        scratch_types=[
            pltpu.VMEM((KB,), jnp.int32),
            pltpu.VMEM((KB, 16), jnp.float32),
            pltpu.VMEM((n // NTILES, 16), jnp.float32),
            pltpu.VMEM_SHARED((n, 16), jnp.float32),
            pltpu.SemaphoreType.DMA,
        ],
    )


def _make_segsum_kernel(n, e, d2):
    ept = e // NTILES
    nb = ept // KB
    assert nb * KB == ept
    rows = n // NTILES
    assert rows * NTILES == n
    rchunk = 125
    nrc = rows // rchunk
    assert nrc * rchunk == rows

    def body(g_hbm, src_hbm, dst_hbm, zero_hbm, out_hbm,
             sidx, didx, msg, obuf, acc, sem):
        cid = lax.axis_index("c")
        sid = lax.axis_index("s")

        @pl.when(sid == 0)
        def _():
            pltpu.sync_copy(zero_hbm, acc)
        plsc.subcore_barrier()

        base0 = sid * ept

        def step(b, carry):
            base = pl.multiple_of(base0 + b * KB, 8)
            pltpu.sync_copy(src_hbm.at[pl.ds(base, KB)], sidx)
            pltpu.sync_copy(dst_hbm.at[pl.ds(base, KB)], didx)
            for j in range(KB // 16):
                sl = pl.ds(j * 16, 16)
                sidx[sl] = sidx[sl] * 2 + cid
            pltpu.async_copy(g_hbm.at[sidx], msg, sem).wait()
            pltpu.sync_copy(msg, acc.at[didx], add=True)
            return carry

        lax.fori_loop(0, nb, step, 0)
        plsc.subcore_barrier()

        for rc in range(nrc):
            r0 = sid * rows + rc * rchunk
            pltpu.sync_copy(acc.at[pl.ds(r0, rchunk)], obuf)
            pltpu.sync_copy(obuf, out_hbm.at[cid, pl.ds(r0, rchunk)])

    return pl.kernel(
        body,
        out_type=jax.ShapeDtypeStruct((NCORES, n, d2), jnp.float32),
        mesh=_mesh(),
        scratch_types=[
            pltpu.VMEM((KB,), jnp.int32),
            pltpu.VMEM((KB,), jnp.int32),
            pltpu.VMEM((KB, d2), jnp.float32),
            pltpu.VMEM((125, d2), jnp.float32),
            pltpu.VMEM_SHARED((n, d2), jnp.float32),
            pltpu.SemaphoreType.DMA,
        ],
    )


def _bn_relu(h, gamma, beta):
    mean = jnp.mean(h, axis=0, keepdims=True)
    var = jnp.mean((h - mean) * (h - mean), axis=0, keepdims=True)
    h = gamma * (h - mean) * lax.rsqrt(var + EPS) + beta
    return jnp.maximum(h, 0.0)


def _dis_from(deg_ref):
    deg = deg_ref[0] + deg_ref[1]
    deg = deg[:, 0:1] + 1.0
    return lax.rsqrt(deg)


def _dense1(x, w1, b1, g0, be0, wc0, degraw):
    n = x.shape[0]
    dh = w1.shape[1]

    def body(x_ref, w1_ref, b1_ref, g_ref, be_ref, wc_ref, deg_ref, out_ref):
        h = jnp.dot(x_ref[...], w1_ref[...],
                    preferred_element_type=jnp.float32) + b1_ref[...]
        h = _bn_relu(h, g_ref[...], be_ref[...])
        hc = jnp.dot(h, wc_ref[...], preferred_element_type=jnp.float32)
        out_ref[...] = hc * _dis_from(deg_ref)

    return pl.pallas_call(
        body,
        out_shape=jax.ShapeDtypeStruct((n, dh), jnp.float32),
    )(x, w1, b1, g0, be0, wc0, degraw)


def _dense_mid(t, gprev, degraw, bc, gbn, bebn, wc):
    n = gprev.shape[0]
    dh = gprev.shape[1]

    def body(t_ref, gp_ref, deg_ref, bc_ref, g_ref, be_ref, wc_ref, out_ref):
        dis = _dis_from(deg_ref)
        t = jnp.concatenate([t_ref[0], t_ref[1]], axis=1)
        h = (t + gp_ref[...]) * dis + bc_ref[...]
        h = _bn_relu(h, g_ref[...], be_ref[...])
        hc = jnp.dot(h, wc_ref[...], preferred_element_type=jnp.float32)
        out_ref[...] = hc * dis

    return pl.pallas_call(
        body,
        out_shape=jax.ShapeDtypeStruct((n, dh), jnp.float32),
    )(t, gprev, degraw, bc, gbn, bebn, wc)


def _dense_out(t, gprev, degraw, bc, gbn, bebn, w2, b2):
    n = gprev.shape[0]
    dout = w2.shape[1]

    def body(t_ref, gp_ref, deg_ref, bc_ref, g_ref, be_ref, w2_ref, b2_ref,
             out_ref):
        dis = _dis_from(deg_ref)
        t = jnp.concatenate([t_ref[0], t_ref[1]], axis=1)
        h = (t + gp_ref[...]) * dis + bc_ref[...]
        h = _bn_relu(h, g_ref[...], be_ref[...])
        out_ref[...] = jnp.dot(h, w2_ref[...],
                               preferred_element_type=jnp.float32) + b2_ref[...]

    return pl.pallas_call(
        body,
        out_shape=jax.ShapeDtypeStruct((n, dout), jnp.float32),
    )(t, gprev, degraw, bc, gbn, bebn, w2, b2)


def kernel(x, adj_t, edge_attr, W_ln1, b_ln1, g_bn0, be_bn0, W_c0, b_c0,
           g_bn1, be_bn1, W_c1, b_c1, g_bn2, be_bn2, W_ln2, b_ln2):
    n = x.shape[0]
    e = adj_t.shape[1]
    dh = W_c0.shape[0]
    d2 = dh // NCORES

    src = adj_t[0].astype(jnp.int32)
    dst = adj_t[1].astype(jnp.int32)

    z16 = jnp.zeros((n, 16), jnp.float32)
    zd2 = jnp.zeros((n, d2), jnp.float32)
    row = lambda v: v.reshape(1, -1)

    deg_k = _make_deg_kernel(n, e)
    seg_k = _make_segsum_kernel(n, e, d2)

    degraw = deg_k(dst, z16)

    g0 = _dense1(x, W_ln1, row(b_ln1), row(g_bn0), row(be_bn0), W_c0, degraw)
    t0 = seg_k(g0.reshape(NCORES * n, d2), src, dst, zd2)

    g1 = _dense_mid(t0, g0, degraw, row(b_c0), row(g_bn1), row(be_bn1), W_c1)
    t1 = seg_k(g1.reshape(NCORES * n, d2), src, dst, zd2)

    return _dense_out(t1, g1, degraw, row(b_c1), row(g_bn2), row(be_bn2),
                      W_ln2, row(b_ln2))


# speedup vs baseline: 8.4238x; 8.4238x over previous
import jax
import jax.numpy as jnp
from jax import lax
from jax.experimental import pallas as pl
from jax.experimental.pallas import tpu as pltpu
from jax.experimental.pallas import tpu_sc as plsc

EPS = 1e-5
NCORES = 2
NTILES = 16
KB = 80
DRC = 128


def _mesh():
    return plsc.VectorSubcoreMesh(
        core_axis_name="c", subcore_axis_name="s",
        num_cores=NCORES, num_subcores=NTILES)


def _stripe(n):
    return -(-n // NTILES // DRC) * DRC


def _make_deg_kernel(n, e):
    epc = e // NCORES
    ept = epc // NTILES
    nb = ept // KB
    assert nb * KB == ept
    stripe = _stripe(n)
    npad = stripe * NTILES
    ndr = stripe // DRC

    def body(dst_hbm, ones_hbm, zero_hbm, out_hbm, didx, ones_v, obuf, acc,
             sem):
        cid = lax.axis_index("c")
        sid = lax.axis_index("s")
        pltpu.sync_copy(ones_hbm, ones_v)
        pltpu.sync_copy(zero_hbm, obuf)
        for i in range(ndr):
            pltpu.sync_copy(obuf, acc.at[pl.ds(sid * stripe + i * DRC, DRC)])
        plsc.subcore_barrier()

        base0 = cid * epc + sid * ept

        def step(b, carry):
            base = pl.multiple_of(base0 + b * KB, 8)
            pltpu.sync_copy(dst_hbm.at[pl.ds(base, KB)], didx)
            pltpu.sync_copy(ones_v, acc.at[didx], add=True)
            return carry

        lax.fori_loop(0, nb, step, 0)
        plsc.subcore_barrier()

        for i in range(ndr):
            r0 = sid * stripe + i * DRC
            pltpu.sync_copy(acc.at[pl.ds(r0, DRC)], obuf)
            pltpu.sync_copy(obuf, out_hbm.at[cid, pl.ds(r0, DRC)])

    return pl.kernel(
        body,
        out_type=jax.ShapeDtypeStruct((NCORES, npad, 128), jnp.float32),
        mesh=_mesh(),
        scratch_types=[
            pltpu.VMEM((KB,), jnp.int32),
            pltpu.VMEM((KB, 128), jnp.float32),
            pltpu.VMEM((DRC, 128), jnp.float32),
            pltpu.VMEM_SHARED((npad, 128), jnp.float32),
            pltpu.SemaphoreType.DMA,
        ],
    )


def _make_segsum_kernel(n, e, dc):
    ept = e // NTILES
    nb = ept // KB
    assert nb * KB == ept
    stripe = _stripe(n)
    npad = stripe * NTILES
    ndr = stripe // DRC

    def body(g_hbm, src_hbm, dst_hbm, zero_hbm, out_hbm,
             sidx, didx, msg, obuf, acc, sem):
        cid = lax.axis_index("c")
        sid = lax.axis_index("s")
        pltpu.sync_copy(zero_hbm, obuf)
        for i in range(ndr):
            pltpu.sync_copy(obuf, acc.at[pl.ds(sid * stripe + i * DRC, DRC)])
        plsc.subcore_barrier()

        base0 = sid * ept

        def step(b, carry):
            base = pl.multiple_of(base0 + b * KB, 8)
            pltpu.sync_copy(src_hbm.at[pl.ds(base, KB)], sidx)
            pltpu.sync_copy(dst_hbm.at[pl.ds(base, KB)], didx)
            for j in range(KB // 16):
                sl = pl.ds(j * 16, 16)
                sidx[sl] = sidx[sl] * NCORES + cid
            pltpu.async_copy(g_hbm.at[sidx], msg, sem).wait()
            pltpu.sync_copy(msg, acc.at[didx], add=True)
            return carry

        lax.fori_loop(0, nb, step, 0)
        plsc.subcore_barrier()

        for i in range(ndr):
            r0 = sid * stripe + i * DRC
            pltpu.sync_copy(acc.at[pl.ds(r0, DRC)], obuf)
            pltpu.sync_copy(obuf, out_hbm.at[cid, pl.ds(r0, DRC)])

    return pl.kernel(
        body,
        out_type=jax.ShapeDtypeStruct((NCORES, npad, dc), jnp.float32),
        mesh=_mesh(),
        scratch_types=[
            pltpu.VMEM((KB,), jnp.int32),
            pltpu.VMEM((KB,), jnp.int32),
            pltpu.VMEM((KB, dc), jnp.float32),
            pltpu.VMEM((DRC, dc), jnp.float32),
            pltpu.VMEM_SHARED((npad, dc), jnp.float32),
            pltpu.SemaphoreType.DMA,
        ],
    )


def _dis_kernel(degraw, n):
    def body(d_ref, o_ref):
        deg = d_ref[0, :n, 0:1] + d_ref[1, :n, 0:1] + 1.0
        o_ref[...] = lax.rsqrt(deg)

    return pl.pallas_call(
        body,
        out_shape=jax.ShapeDtypeStruct((n, 1), jnp.float32),
    )(degraw)


def _bn_relu(h, gamma, beta):
    mean = jnp.mean(h, axis=0, keepdims=True)
    var = jnp.mean((h - mean) * (h - mean), axis=0, keepdims=True)
    h = gamma * (h - mean) * lax.rsqrt(var + EPS) + beta
    return jnp.maximum(h, 0.0)


def _dense1(x, w1, b1, g0, be0, wc0, dis):
    n = x.shape[0]
    dh = w1.shape[1]

    def body(x_ref, w1_ref, b1_ref, g_ref, be_ref, wc_ref, dis_ref, out_ref):
        h = jnp.dot(x_ref[...], w1_ref[...],
                    preferred_element_type=jnp.float32) + b1_ref[...]
        h = _bn_relu(h, g_ref[...], be_ref[...])
        hc = jnp.dot(h, wc_ref[...], preferred_element_type=jnp.float32)
        out_ref[...] = hc * dis_ref[...]

    return pl.pallas_call(
        body,
        out_shape=jax.ShapeDtypeStruct((n, dh), jnp.float32),
    )(x, w1, b1, g0, be0, wc0, dis)


def _dense_step(t, gprev, dis, bc, gbn, bebn, w, b2, u):
    n = gprev.shape[0]
    dh = w.shape[1]

    def body(t_ref, gp_ref, dis_ref, bc_ref, g_ref, be_ref, w_ref, b2_ref,
             u_ref, out_ref):
        dis = dis_ref[...]
        nt = t_ref.shape[0]
        t = jnp.concatenate([t_ref[c, :n, :] for c in range(nt)], axis=1)
        h = (t + gp_ref[...]) * dis + bc_ref[...]
        h = _bn_relu(h, g_ref[...], be_ref[...])
        hc = jnp.dot(h, w_ref[...], preferred_element_type=jnp.float32)
        u = u_ref[0, 0]
        out_ref[...] = hc * (dis * u + (1.0 - u)) + b2_ref[...]

    return pl.pallas_call(
        body,
        out_shape=jax.ShapeDtypeStruct((n, dh), jnp.float32),
    )(t, gprev, dis, bc, gbn, bebn, w, b2, u)


def kernel(x, adj_t, edge_attr, W_ln1, b_ln1, g_bn0, be_bn0, W_c0, b_c0,
           g_bn1, be_bn1, W_c1, b_c1, g_bn2, be_bn2, W_ln2, b_ln2):
    n = x.shape[0]
    e = adj_t.shape[1]
    dh = W_c0.shape[0]
    dc = dh // NCORES

    src = adj_t[0].astype(jnp.int32)
    dst = adj_t[1].astype(jnp.int32)

    ones128 = jnp.ones((KB, 128), jnp.float32)
    z128 = jnp.zeros((DRC, 128), jnp.float32)
    row = lambda v: v.reshape(1, -1)

    deg_k = _make_deg_kernel(n, e)
    seg_k = _make_segsum_kernel(n, e, dc)

    degraw = deg_k(dst, ones128, z128)
    dis = _dis_kernel(degraw, n)

    g0 = _dense1(x, W_ln1, row(b_ln1), row(g_bn0), row(be_bn0), W_c0, dis)

    t0 = seg_k(g0.reshape(NCORES * n, dc), src, dst, z128)
    g1 = _dense_step(t0, g0, dis, row(b_c0), row(g_bn1), row(be_bn1),
                     W_c1, jnp.zeros((1, dh), jnp.float32),
                     jnp.ones((1, 1), jnp.float32))

    t1 = seg_k(g1.reshape(NCORES * n, dc), src, dst, z128)
    return _dense_step(t1, g1, dis, row(b_c1), row(g_bn2), row(be_bn2),
                       W_ln2, row(b_ln2), jnp.zeros((1, 1), jnp.float32))


# speedup vs baseline: 11.0169x; 1.3078x over previous
import jax
import jax.numpy as jnp
from jax import lax
from jax.experimental import pallas as pl
from jax.experimental.pallas import tpu as pltpu
from jax.experimental.pallas import tpu_sc as plsc

EPS = 1e-5
NCORES = 2
NTILES = 16
KB = 128
KBD = 40
DRC = 64


def _mesh():
    return plsc.VectorSubcoreMesh(
        core_axis_name="c", subcore_axis_name="s",
        num_cores=NCORES, num_subcores=NTILES)


def _stripe(n):
    return -(-n // NTILES // DRC) * DRC


def _make_deg_kernel(n, e):
    epc = e // NCORES
    ept = epc // NTILES
    nb = ept // KBD
    assert nb * KBD == ept and nb % 2 == 0 and nb >= 4
    stripe = _stripe(n)
    npad = stripe * NTILES
    ndr = stripe // DRC

    def body(dst_hbm, ones_hbm, zero_hbm, out_hbm, didx0, didx1, ones_v,
             obuf, acc, sem0, sem1):
        cid = lax.axis_index("c")
        sid = lax.axis_index("s")
        dbufs, sems = (didx0, didx1), (sem0, sem1)
        pltpu.sync_copy(ones_hbm, ones_v)
        pltpu.sync_copy(zero_hbm, obuf)
        for i in range(ndr):
            pltpu.sync_copy(obuf, acc.at[pl.ds(sid * stripe + i * DRC, DRC)])
        plsc.subcore_barrier()

        base0 = cid * epc + sid * ept

        def start(b, k):
            base = pl.multiple_of(base0 + b * KBD, 8)
            pltpu.async_copy(dst_hbm.at[pl.ds(base, KBD)], dbufs[k], sems[k])

        def finish(k):
            pltpu.make_async_copy(dst_hbm.at[pl.ds(0, KBD)], dbufs[k],
                                  sems[k]).wait()
            pltpu.sync_copy(ones_v, acc.at[dbufs[k]], add=True)

        start(0, 0)
        start(1, 1)

        def step(j, carry):
            finish(0)
            start(2 * j + 2, 0)
            finish(1)
            start(2 * j + 3, 1)
            return carry

        lax.fori_loop(0, (nb - 2) // 2, step, 0)
        finish(0)
        finish(1)
        plsc.subcore_barrier()

        for i in range(ndr):
            r0 = sid * stripe + i * DRC
            pltpu.sync_copy(acc.at[pl.ds(r0, DRC)], obuf)
            pltpu.sync_copy(obuf, out_hbm.at[cid, pl.ds(r0, DRC)])

    return pl.kernel(
        body,
        out_type=jax.ShapeDtypeStruct((NCORES, npad, 128), jnp.float32),
        mesh=_mesh(),
        scratch_types=[
            pltpu.VMEM((KBD,), jnp.int32),
            pltpu.VMEM((KBD,), jnp.int32),
            pltpu.VMEM((KBD, 128), jnp.float32),
            pltpu.VMEM((DRC, 128), jnp.float32),
            pltpu.VMEM_SHARED((npad, 128), jnp.float32),
            pltpu.SemaphoreType.DMA,
            pltpu.SemaphoreType.DMA,
        ],
    )


def _make_segsum_kernel(n, e, dc):
    ept = e // NTILES
    nb = ept // KB
    assert nb * KB == ept and nb % 2 == 0 and nb >= 4
    stripe = _stripe(n)
    npad = stripe * NTILES
    ndr = stripe // DRC

    def body(g_hbm, src_hbm, dst_hbm, zero_hbm, out_hbm,
             sidx0, sidx1, didx0, didx1, msg0, msg1, obuf, acc,
             sem0, sem1):
        cid = lax.axis_index("c")
        sid = lax.axis_index("s")
        sbufs, dbufs = (sidx0, sidx1), (didx0, didx1)
        msgs, sems = (msg0, msg1), (sem0, sem1)

        pltpu.sync_copy(zero_hbm, obuf)
        for i in range(ndr):
            pltpu.sync_copy(obuf, acc.at[pl.ds(sid * stripe + i * DRC, DRC)])
        plsc.subcore_barrier()

        base0 = sid * ept

        def load_and_start(b, k):
            base = pl.multiple_of(base0 + b * KB, 8)
            pltpu.sync_copy(src_hbm.at[pl.ds(base, KB)], sbufs[k])
            pltpu.sync_copy(dst_hbm.at[pl.ds(base, KB)], dbufs[k])
            for j in range(KB // 16):
                sl = pl.ds(j * 16, 16)
                sbufs[k][sl] = sbufs[k][sl] * NCORES + cid
            pltpu.async_copy(g_hbm.at[sbufs[k]], msgs[k], sems[k])

        def finish(k):
            pltpu.make_async_copy(g_hbm.at[sbufs[k]], msgs[k], sems[k]).wait()
            pltpu.sync_copy(msgs[k], acc.at[dbufs[k]], add=True)

        load_and_start(0, 0)
        load_and_start(1, 1)

        def step(j, carry):
            finish(0)
            load_and_start(2 * j + 2, 0)
            finish(1)
            load_and_start(2 * j + 3, 1)
            return carry

        lax.fori_loop(0, (nb - 2) // 2, step, 0)
        finish(0)
        finish(1)
        plsc.subcore_barrier()

        for i in range(ndr):
            r0 = sid * stripe + i * DRC
            pltpu.sync_copy(acc.at[pl.ds(r0, DRC)], obuf)
            pltpu.sync_copy(obuf, out_hbm.at[cid, pl.ds(r0, DRC)])

    return pl.kernel(
        body,
        out_type=jax.ShapeDtypeStruct((NCORES, npad, dc), jnp.float32),
        mesh=_mesh(),
        scratch_types=[
            pltpu.VMEM((KB,), jnp.int32),
            pltpu.VMEM((KB,), jnp.int32),
            pltpu.VMEM((KB,), jnp.int32),
            pltpu.VMEM((KB,), jnp.int32),
            pltpu.VMEM((KB, dc), jnp.float32),
            pltpu.VMEM((KB, dc), jnp.float32),
            pltpu.VMEM((DRC, dc), jnp.float32),
            pltpu.VMEM_SHARED((npad, dc), jnp.float32),
            pltpu.SemaphoreType.DMA,
            pltpu.SemaphoreType.DMA,
        ],
    )


def _dis_kernel(degraw, n):
    def body(d_ref, o_ref):
        deg = d_ref[0, :n, 0:1] + d_ref[1, :n, 0:1] + 1.0
        o_ref[...] = lax.rsqrt(deg)

    return pl.pallas_call(
        body,
        out_shape=jax.ShapeDtypeStruct((n, 1), jnp.float32),
    )(degraw)


def _bn_relu(h, gamma, beta):
    mean = jnp.mean(h, axis=0, keepdims=True)
    var = jnp.mean((h - mean) * (h - mean), axis=0, keepdims=True)
    h = gamma * (h - mean) * lax.rsqrt(var + EPS) + beta
    return jnp.maximum(h, 0.0)


def _dense1(x, w1, b1, g0, be0, wc0, dis):
    n = x.shape[0]
    dh = w1.shape[1]

    def body(x_ref, w1_ref, b1_ref, g_ref, be_ref, wc_ref, dis_ref, out_ref):
        h = jnp.dot(x_ref[...], w1_ref[...],
                    preferred_element_type=jnp.float32) + b1_ref[...]
        h = _bn_relu(h, g_ref[...], be_ref[...])
        hc = jnp.dot(h, wc_ref[...], preferred_element_type=jnp.float32)
        out_ref[...] = hc * dis_ref[...]

    return pl.pallas_call(
        body,
        out_shape=jax.ShapeDtypeStruct((n, dh), jnp.float32),
    )(x, w1, b1, g0, be0, wc0, dis)


def _dense_step(t, gprev, dis, bc, gbn, bebn, w, b2, u):
    n = gprev.shape[0]
    dh = w.shape[1]

    def body(t_ref, gp_ref, dis_ref, bc_ref, g_ref, be_ref, w_ref, b2_ref,
             u_ref, out_ref):
        dis = dis_ref[...]
        nt = t_ref.shape[0]
        t = jnp.concatenate([t_ref[c, :n, :] for c in range(nt)], axis=1)
        h = (t + gp_ref[...]) * dis + bc_ref[...]
        h = _bn_relu(h, g_ref[...], be_ref[...])
        hc = jnp.dot(h, w_ref[...], preferred_element_type=jnp.float32)
        u = u_ref[0, 0]
        out_ref[...] = hc * (dis * u + (1.0 - u)) + b2_ref[...]

    return pl.pallas_call(
        body,
        out_shape=jax.ShapeDtypeStruct((n, dh), jnp.float32),
    )(t, gprev, dis, bc, gbn, bebn, w, b2, u)


def kernel(x, adj_t, edge_attr, W_ln1, b_ln1, g_bn0, be_bn0, W_c0, b_c0,
           g_bn1, be_bn1, W_c1, b_c1, g_bn2, be_bn2, W_ln2, b_ln2):
    n = x.shape[0]
    e = adj_t.shape[1]
    dh = W_c0.shape[0]
    dc = dh // NCORES

    src = adj_t[0].astype(jnp.int32)
    dst = adj_t[1].astype(jnp.int32)

    stripe = _stripe(n)
    npad = stripe * NTILES
    nb = -(-e // (NTILES * KB))
    nb += nb % 2
    e_pad = NTILES * KB * nb
    pad = e_pad - e
    srcp = jnp.concatenate([src, jnp.zeros((pad,), jnp.int32)])
    dstp = jnp.concatenate(
        [dst, n + (jnp.arange(pad, dtype=jnp.int32) % (npad - n))])

    ones128 = jnp.ones((KBD, 128), jnp.float32)
    z128 = jnp.zeros((DRC, 128), jnp.float32)
    row = lambda v: v.reshape(1, -1)

    deg_k = _make_deg_kernel(n, e)
    seg_k = _make_segsum_kernel(n, e_pad, dc)

    degraw = deg_k(dst, ones128, z128)
    dis = _dis_kernel(degraw, n)

    g0 = _dense1(x, W_ln1, row(b_ln1), row(g_bn0), row(be_bn0), W_c0, dis)

    t0 = seg_k(g0.reshape(NCORES * n, dc), srcp, dstp, z128)
    g1 = _dense_step(t0, g0, dis, row(b_c0), row(g_bn1), row(be_bn1),
                     W_c1, jnp.zeros((1, dh), jnp.float32),
                     jnp.ones((1, 1), jnp.float32))

    t1 = seg_k(g1.reshape(NCORES * n, dc), srcp, dstp, z128)
    return _dense_step(t1, g1, dis, row(b_c1), row(g_bn2), row(be_bn2),
                       W_ln2, row(b_ln2), jnp.zeros((1, 1), jnp.float32))


# speedup vs baseline: 13.2597x; 1.2036x over previous
import jax
import jax.numpy as jnp
from jax import lax
from jax.experimental import pallas as pl
from jax.experimental.pallas import tpu as pltpu
from jax.experimental.pallas import tpu_sc as plsc

EPS = 1e-5
NCORES = 2
NTILES = 16
KB = 80
KBD = 40
DRC = 64


def _mesh():
    return plsc.VectorSubcoreMesh(
        core_axis_name="c", subcore_axis_name="s",
        num_cores=NCORES, num_subcores=NTILES)


def _stripe(n):
    return -(-n // NTILES // DRC) * DRC


def _make_deg_kernel(n, e):
    epc = e // NCORES
    ept = epc // NTILES
    nb = ept // KBD
    assert nb * KBD == ept and nb % 2 == 0 and nb >= 4
    stripe = _stripe(n)
    npad = stripe * NTILES
    ndr = stripe // DRC

    def body(dst_hbm, ones_hbm, zero_hbm, out_hbm, didx0, didx1, ones_v,
             obuf, acc, sem0, sem1):
        cid = lax.axis_index("c")
        sid = lax.axis_index("s")
        dbufs, sems = (didx0, didx1), (sem0, sem1)
        pltpu.sync_copy(ones_hbm, ones_v)
        pltpu.sync_copy(zero_hbm, obuf)
        for i in range(ndr):
            pltpu.sync_copy(obuf, acc.at[pl.ds(sid * stripe + i * DRC, DRC)])
        plsc.subcore_barrier()

        base0 = cid * epc + sid * ept

        def start(b, k):
            base = pl.multiple_of(base0 + b * KBD, 8)
            pltpu.async_copy(dst_hbm.at[pl.ds(base, KBD)], dbufs[k], sems[k])

        def finish(k):
            pltpu.make_async_copy(dst_hbm.at[pl.ds(0, KBD)], dbufs[k],
                                  sems[k]).wait()
            pltpu.sync_copy(ones_v, acc.at[dbufs[k]], add=True)

        start(0, 0)
        start(1, 1)

        def step(j, carry):
            finish(0)
            start(2 * j + 2, 0)
            finish(1)
            start(2 * j + 3, 1)
            return carry

        lax.fori_loop(0, (nb - 2) // 2, step, 0)
        finish(0)
        finish(1)
        plsc.subcore_barrier()

        for i in range(ndr):
            r0 = sid * stripe + i * DRC
            pltpu.sync_copy(acc.at[pl.ds(r0, DRC)], obuf)
            pltpu.sync_copy(obuf, out_hbm.at[cid, pl.ds(r0, DRC)])

    return pl.kernel(
        body,
        out_type=jax.ShapeDtypeStruct((NCORES, npad, 128), jnp.float32),
        mesh=_mesh(),
        scratch_types=[
            pltpu.VMEM((KBD,), jnp.int32),
            pltpu.VMEM((KBD,), jnp.int32),
            pltpu.VMEM((KBD, 128), jnp.float32),
            pltpu.VMEM((DRC, 128), jnp.float32),
            pltpu.VMEM_SHARED((npad, 128), jnp.float32),
            pltpu.SemaphoreType.DMA,
            pltpu.SemaphoreType.DMA,
        ],
    )


def _make_segsum_kernel(n, e, dc):
    ept = e // NTILES
    nb = ept // KB
    assert nb * KB == ept and nb % 2 == 0 and nb >= 4
    stripe = _stripe(n)
    npad = stripe * NTILES
    ndr = stripe // DRC

    def body(g_hbm, src_hbm, dst_hbm, zero_hbm, out_hbm,
             sidx0, sidx1, didx0, didx1, msg0, msg1, obuf, acc,
             sem0, sem1):
        cid = lax.axis_index("c")
        sid = lax.axis_index("s")
        sbufs, dbufs = (sidx0, sidx1), (didx0, didx1)
        msgs, sems = (msg0, msg1), (sem0, sem1)

        pltpu.sync_copy(zero_hbm, obuf)
        for i in range(ndr):
            pltpu.sync_copy(obuf, acc.at[pl.ds(sid * stripe + i * DRC, DRC)])
        plsc.subcore_barrier()

        base0 = sid * ept

        def load_and_start(b, k):
            base = pl.multiple_of(base0 + b * KB, 8)
            pltpu.sync_copy(src_hbm.at[pl.ds(base, KB)], sbufs[k])
            pltpu.sync_copy(dst_hbm.at[pl.ds(base, KB)], dbufs[k])
            for j in range(KB // 16):
                sl = pl.ds(j * 16, 16)
                sbufs[k][sl] = sbufs[k][sl] * NCORES + cid
            pltpu.async_copy(g_hbm.at[sbufs[k]], msgs[k], sems[k])

        def finish(k):
            pltpu.make_async_copy(g_hbm.at[sbufs[k]], msgs[k], sems[k]).wait()
            pltpu.sync_copy(msgs[k], acc.at[dbufs[k]], add=True)

        load_and_start(0, 0)
        load_and_start(1, 1)

        def step(j, carry):
            finish(0)
            load_and_start(2 * j + 2, 0)
            finish(1)
            load_and_start(2 * j + 3, 1)
            return carry

        lax.fori_loop(0, (nb - 2) // 2, step, 0)
        finish(0)
        finish(1)
        plsc.subcore_barrier()

        for i in range(ndr):
            r0 = sid * stripe + i * DRC
            pltpu.sync_copy(acc.at[pl.ds(r0, DRC)], obuf)
            pltpu.sync_copy(obuf, out_hbm.at[cid, pl.ds(r0, DRC)])

    return pl.kernel(
        body,
        out_type=jax.ShapeDtypeStruct((NCORES, npad, dc), jnp.float32),
        mesh=_mesh(),
        scratch_types=[
            pltpu.VMEM((KB,), jnp.int32),
            pltpu.VMEM((KB,), jnp.int32),
            pltpu.VMEM((KB,), jnp.int32),
            pltpu.VMEM((KB,), jnp.int32),
            pltpu.VMEM((KB, dc), jnp.float32),
            pltpu.VMEM((KB, dc), jnp.float32),
            pltpu.VMEM((DRC, dc), jnp.float32),
            pltpu.VMEM_SHARED((npad, dc), jnp.float32),
            pltpu.SemaphoreType.DMA,
            pltpu.SemaphoreType.DMA,
        ],
    )


def _dis_kernel(degraw, n):
    def body(d_ref, o_ref):
        deg = d_ref[0, :n, 0:1] + d_ref[1, :n, 0:1] + 1.0
        o_ref[...] = lax.rsqrt(deg)

    return pl.pallas_call(
        body,
        out_shape=jax.ShapeDtypeStruct((n, 1), jnp.float32),
    )(degraw)


def _bn_relu(h, gamma, beta):
    mean = jnp.mean(h, axis=0, keepdims=True)
    var = jnp.mean((h - mean) * (h - mean), axis=0, keepdims=True)
    h = gamma * (h - mean) * lax.rsqrt(var + EPS) + beta
    return jnp.maximum(h, 0.0)


def _dense1(x, w1, b1, g0, be0, wc0, dis):
    n = x.shape[0]
    dh = w1.shape[1]

    def body(x_ref, w1_ref, b1_ref, g_ref, be_ref, wc_ref, dis_ref, out_ref):
        h = jnp.dot(x_ref[...], w1_ref[...],
                    preferred_element_type=jnp.float32) + b1_ref[...]
        h = _bn_relu(h, g_ref[...], be_ref[...])
        hc = jnp.dot(h, wc_ref[...], preferred_element_type=jnp.float32)
        out_ref[...] = hc * dis_ref[...]

    return pl.pallas_call(
        body,
        out_shape=jax.ShapeDtypeStruct((n, dh), jnp.float32),
    )(x, w1, b1, g0, be0, wc0, dis)


def _dense_step(t, gprev, dis, bc, gbn, bebn, w, b2, u):
    n = gprev.shape[0]
    dh = w.shape[1]

    def body(t_ref, gp_ref, dis_ref, bc_ref, g_ref, be_ref, w_ref, b2_ref,
             u_ref, out_ref):
        dis = dis_ref[...]
        nt = t_ref.shape[0]
        t = jnp.concatenate([t_ref[c, :n, :] for c in range(nt)], axis=1)
        h = (t + gp_ref[...]) * dis + bc_ref[...]
        h = _bn_relu(h, g_ref[...], be_ref[...])
        hc = jnp.dot(h, w_ref[...], preferred_element_type=jnp.float32)
        u = u_ref[0, 0]
        out_ref[...] = hc * (dis * u + (1.0 - u)) + b2_ref[...]

    return pl.pallas_call(
        body,
        out_shape=jax.ShapeDtypeStruct((n, dh), jnp.float32),
    )(t, gprev, dis, bc, gbn, bebn, w, b2, u)


def kernel(x, adj_t, edge_attr, W_ln1, b_ln1, g_bn0, be_bn0, W_c0, b_c0,
           g_bn1, be_bn1, W_c1, b_c1, g_bn2, be_bn2, W_ln2, b_ln2):
    n = x.shape[0]
    e = adj_t.shape[1]
    dh = W_c0.shape[0]
    dc = dh // NCORES

    src = adj_t[0].astype(jnp.int32)
    dst = adj_t[1].astype(jnp.int32)

    stripe = _stripe(n)
    npad = stripe * NTILES
    nb = -(-e // (NTILES * KB))
    nb += nb % 2
    e_pad = NTILES * KB * nb
    pad = e_pad - e
    srcp = jnp.concatenate([src, jnp.zeros((pad,), jnp.int32)])
    dstp = jnp.concatenate(
        [dst, n + (jnp.arange(pad, dtype=jnp.int32) % (npad - n))])

    ones128 = jnp.ones((KBD, 128), jnp.float32)
    z128 = jnp.zeros((DRC, 128), jnp.float32)
    row = lambda v: v.reshape(1, -1)

    deg_k = _make_deg_kernel(n, e)
    seg_k = _make_segsum_kernel(n, e_pad, dc)

    degraw = deg_k(dst, ones128, z128)
    dis = _dis_kernel(degraw, n)

    g0 = _dense1(x, W_ln1, row(b_ln1), row(g_bn0), row(be_bn0), W_c0, dis)

    t0 = seg_k(g0.reshape(NCORES * n, dc), srcp, dstp, z128)
    g1 = _dense_step(t0, g0, dis, row(b_c0), row(g_bn1), row(be_bn1),
                     W_c1, jnp.zeros((1, dh), jnp.float32),
                     jnp.ones((1, 1), jnp.float32))

    t1 = seg_k(g1.reshape(NCORES * n, dc), srcp, dstp, z128)
    return _dense_step(t1, g1, dis, row(b_c1), row(g_bn2), row(be_bn2),
                       W_ln2, row(b_ln2), jnp.zeros((1, 1), jnp.float32))


# speedup vs baseline: 13.5816x; 1.0243x over previous
import jax
import jax.numpy as jnp
from jax import lax
from jax.experimental import pallas as pl
from jax.experimental.pallas import tpu as pltpu
from jax.experimental.pallas import tpu_sc as plsc

EPS = 1e-5
NCORES = 2
NTILES = 16
KB = 80
KBD = 40
DRC = 64


def _mesh():
    return plsc.VectorSubcoreMesh(
        core_axis_name="c", subcore_axis_name="s",
        num_cores=NCORES, num_subcores=NTILES)


def _stripe(n):
    return -(-n // NTILES // DRC) * DRC


def _make_deg_kernel(n, e):
    epc = e // NCORES
    ept = epc // NTILES
    nb = ept // KBD
    assert nb * KBD == ept and nb % 2 == 0 and nb >= 4
    stripe = _stripe(n)
    npad = stripe * NTILES
    ndr = stripe // DRC

    def body(dst_hbm, ones_hbm, zero_hbm, out_hbm, didx0, didx1, ones_v,
             obuf, acc, sem0, sem1):
        cid = lax.axis_index("c")
        sid = lax.axis_index("s")
        dbufs, sems = (didx0, didx1), (sem0, sem1)
        pltpu.sync_copy(ones_hbm, ones_v)
        pltpu.sync_copy(zero_hbm, obuf)
        for i in range(ndr):
            pltpu.sync_copy(obuf, acc.at[pl.ds(sid * stripe + i * DRC, DRC)])
        plsc.subcore_barrier()

        base0 = cid * epc + sid * ept

        def start(b, k):
            base = pl.multiple_of(base0 + b * KBD, 8)
            pltpu.async_copy(dst_hbm.at[pl.ds(base, KBD)], dbufs[k], sems[k])

        def finish(k):
            pltpu.make_async_copy(dst_hbm.at[pl.ds(0, KBD)], dbufs[k],
                                  sems[k]).wait()
            pltpu.sync_copy(ones_v, acc.at[dbufs[k]], add=True)

        start(0, 0)
        start(1, 1)

        def step(j, carry):
            finish(0)
            start(2 * j + 2, 0)
            finish(1)
            start(2 * j + 3, 1)
            return carry

        lax.fori_loop(0, (nb - 2) // 2, step, 0)
        finish(0)
        finish(1)
        plsc.subcore_barrier()

        for i in range(ndr):
            r0 = sid * stripe + i * DRC
            pltpu.sync_copy(acc.at[pl.ds(r0, DRC)], obuf)
            pltpu.sync_copy(obuf, out_hbm.at[cid, pl.ds(r0, DRC)])

    return pl.kernel(
        body,
        out_type=jax.ShapeDtypeStruct((NCORES, npad, 128), jnp.float32),
        mesh=_mesh(),
        scratch_types=[
            pltpu.VMEM((KBD,), jnp.int32),
            pltpu.VMEM((KBD,), jnp.int32),
            pltpu.VMEM((KBD, 128), jnp.float32),
            pltpu.VMEM((DRC, 128), jnp.float32),
            pltpu.VMEM_SHARED((npad, 128), jnp.float32),
            pltpu.SemaphoreType.DMA,
            pltpu.SemaphoreType.DMA,
        ],
    )


def _make_segsum_kernel(n, e, dc):
    ept = e // NTILES
    nb = ept // KB
    assert nb * KB == ept and nb % 2 == 0 and nb >= 4
    stripe = _stripe(n)
    npad = stripe * NTILES
    ndr = stripe // DRC

    assert nb % 4 == 0 and nb >= 8

    def body(g_hbm, src_hbm, dst_hbm, zero_hbm, out_hbm,
             sidx0, sidx1, sidx2, sidx3, didx0, didx1, didx2, didx3,
             msg0, msg1, obuf, acc,
             semi0, semi1, semi2, semi3, semg0, semg1):
        cid = lax.axis_index("c")
        sid = lax.axis_index("s")
        sbufs = (sidx0, sidx1, sidx2, sidx3)
        dbufs = (didx0, didx1, didx2, didx3)
        msgs, semg = (msg0, msg1), (semg0, semg1)
        semi = (semi0, semi1, semi2, semi3)

        pltpu.sync_copy(zero_hbm, obuf)
        for i in range(ndr):
            pltpu.sync_copy(obuf, acc.at[pl.ds(sid * stripe + i * DRC, DRC)])
        plsc.subcore_barrier()

        base0 = sid * ept

        def idx_start(b, k):
            base = pl.multiple_of(base0 + b * KB, 8)
            pltpu.async_copy(src_hbm.at[pl.ds(base, KB)], sbufs[k], semi[k])
            pltpu.async_copy(dst_hbm.at[pl.ds(base, KB)], dbufs[k], semi[k])

        def gather_go(k, m):
            z = pl.ds(0, KB)
            pltpu.make_async_copy(src_hbm.at[z], sbufs[k], semi[k]).wait()
            pltpu.make_async_copy(dst_hbm.at[z], dbufs[k], semi[k]).wait()
            for j in range(KB // 16):
                sl = pl.ds(j * 16, 16)
                sbufs[k][sl] = sbufs[k][sl] * NCORES + cid
            pltpu.async_copy(g_hbm.at[sbufs[k]], msgs[m], semg[m])

        def finish(m, k):
            pltpu.make_async_copy(g_hbm.at[sbufs[k]], msgs[m], semg[m]).wait()
            pltpu.sync_copy(msgs[m], acc.at[dbufs[k]], add=True)

        for k in range(4):
            idx_start(k, k)
        gather_go(0, 0)
        gather_go(1, 1)

        def step(j, carry):
            b = 4 * j
            finish(0, 0)
            idx_start(b + 4, 0)
            gather_go(2, 0)
            finish(1, 1)
            idx_start(b + 5, 1)
            gather_go(3, 1)
            finish(0, 2)
            idx_start(b + 6, 2)
            gather_go(0, 0)
            finish(1, 3)
            idx_start(b + 7, 3)
            gather_go(1, 1)
            return carry

        lax.fori_loop(0, (nb - 4) // 4, step, 0)
        finish(0, 0)
        finish(1, 1)
        gather_go(2, 0)
        gather_go(3, 1)
        finish(0, 2)
        finish(1, 3)
        plsc.subcore_barrier()

        for i in range(ndr):
            r0 = sid * stripe + i * DRC
            pltpu.sync_copy(acc.at[pl.ds(r0, DRC)], obuf)
            pltpu.sync_copy(obuf, out_hbm.at[cid, pl.ds(r0, DRC)])

    return pl.kernel(
        body,
        out_type=jax.ShapeDtypeStruct((NCORES, npad, dc), jnp.float32),
        mesh=_mesh(),
        scratch_types=[
            pltpu.VMEM((KB,), jnp.int32),
            pltpu.VMEM((KB,), jnp.int32),
            pltpu.VMEM((KB,), jnp.int32),
            pltpu.VMEM((KB,), jnp.int32),
            pltpu.VMEM((KB,), jnp.int32),
            pltpu.VMEM((KB,), jnp.int32),
            pltpu.VMEM((KB,), jnp.int32),
            pltpu.VMEM((KB,), jnp.int32),
            pltpu.VMEM((KB, dc), jnp.float32),
            pltpu.VMEM((KB, dc), jnp.float32),
            pltpu.VMEM((DRC, dc), jnp.float32),
            pltpu.VMEM_SHARED((npad, dc), jnp.float32),
            pltpu.SemaphoreType.DMA,
            pltpu.SemaphoreType.DMA,
            pltpu.SemaphoreType.DMA,
            pltpu.SemaphoreType.DMA,
            pltpu.SemaphoreType.DMA,
            pltpu.SemaphoreType.DMA,
        ],
    )


def _dis_kernel(degraw, n):
    def body(d_ref, o_ref):
        deg = d_ref[0, :n, 0:1] + d_ref[1, :n, 0:1] + 1.0
        o_ref[...] = lax.rsqrt(deg)

    return pl.pallas_call(
        body,
        out_shape=jax.ShapeDtypeStruct((n, 1), jnp.float32),
    )(degraw)


def _bn_relu(h, gamma, beta):
    mean = jnp.mean(h, axis=0, keepdims=True)
    var = jnp.mean((h - mean) * (h - mean), axis=0, keepdims=True)
    h = gamma * (h - mean) * lax.rsqrt(var + EPS) + beta
    return jnp.maximum(h, 0.0)


def _dense1(x, w1, b1, g0, be0, wc0, dis):
    n = x.shape[0]
    dh = w1.shape[1]

    def body(x_ref, w1_ref, b1_ref, g_ref, be_ref, wc_ref, dis_ref, out_ref):
        h = jnp.dot(x_ref[...], w1_ref[...],
                    preferred_element_type=jnp.float32) + b1_ref[...]
        h = _bn_relu(h, g_ref[...], be_ref[...])
        hc = jnp.dot(h, wc_ref[...], preferred_element_type=jnp.float32)
        out_ref[...] = hc * dis_ref[...]

    return pl.pallas_call(
        body,
        out_shape=jax.ShapeDtypeStruct((n, dh), jnp.float32),
    )(x, w1, b1, g0, be0, wc0, dis)


def _dense_step(t, gprev, dis, bc, gbn, bebn, w, b2, u):
    n = gprev.shape[0]
    dh = w.shape[1]

    def body(t_ref, gp_ref, dis_ref, bc_ref, g_ref, be_ref, w_ref, b2_ref,
             u_ref, out_ref):
        dis = dis_ref[...]
        nt = t_ref.shape[0]
        t = jnp.concatenate([t_ref[c, :n, :] for c in range(nt)], axis=1)
        h = (t + gp_ref[...]) * dis + bc_ref[...]
        h = _bn_relu(h, g_ref[...], be_ref[...])
        hc = jnp.dot(h, w_ref[...], preferred_element_type=jnp.float32)
        u = u_ref[0, 0]
        out_ref[...] = hc * (dis * u + (1.0 - u)) + b2_ref[...]

    return pl.pallas_call(
        body,
        out_shape=jax.ShapeDtypeStruct((n, dh), jnp.float32),
    )(t, gprev, dis, bc, gbn, bebn, w, b2, u)


def kernel(x, adj_t, edge_attr, W_ln1, b_ln1, g_bn0, be_bn0, W_c0, b_c0,
           g_bn1, be_bn1, W_c1, b_c1, g_bn2, be_bn2, W_ln2, b_ln2):
    n = x.shape[0]
    e = adj_t.shape[1]
    dh = W_c0.shape[0]
    dc = dh // NCORES

    src = adj_t[0].astype(jnp.int32)
    dst = adj_t[1].astype(jnp.int32)

    stripe = _stripe(n)
    npad = stripe * NTILES
    nb = (-(-e // (NTILES * KB)) + 3) // 4 * 4
    e_pad = NTILES * KB * nb
    pad = e_pad - e
    srcp = jnp.concatenate([src, jnp.zeros((pad,), jnp.int32)])
    dstp = jnp.concatenate(
        [dst, n + (jnp.arange(pad, dtype=jnp.int32) % (npad - n))])

    ones128 = jnp.ones((KBD, 128), jnp.float32)
    z128 = jnp.zeros((DRC, 128), jnp.float32)
    row = lambda v: v.reshape(1, -1)

    deg_k = _make_deg_kernel(n, e)
    seg_k = _make_segsum_kernel(n, e_pad, dc)

    degraw = deg_k(dst, ones128, z128)
    dis = _dis_kernel(degraw, n)

    g0 = _dense1(x, W_ln1, row(b_ln1), row(g_bn0), row(be_bn0), W_c0, dis)

    t0 = seg_k(g0.reshape(NCORES * n, dc), srcp, dstp, z128)
    g1 = _dense_step(t0, g0, dis, row(b_c0), row(g_bn1), row(be_bn1),
                     W_c1, jnp.zeros((1, dh), jnp.float32),
                     jnp.ones((1, 1), jnp.float32))

    t1 = seg_k(g1.reshape(NCORES * n, dc), srcp, dstp, z128)
    return _dense_step(t1, g1, dis, row(b_c1), row(g_bn2), row(be_bn2),
                       W_ln2, row(b_ln2), jnp.zeros((1, 1), jnp.float32))
